# Initial kernel scaffold; baseline (speedup 1.0000x reference)
import jax
import jax.numpy as jnp
from jax.experimental import pallas as pl


def kernel(x, batch_idx, src, dst):
    raise NotImplementedError("write your pallas kernel here")


# speedup vs baseline: 2.0005x; 2.0005x over previous
import functools

import jax
import jax.numpy as jnp
from jax import lax
from jax.experimental import pallas as pl
from jax.experimental.pallas import tpu as pltpu
from jax.experimental.pallas import tpu_sc as plsc

B, N, D = 4, 10000, 128
E = 500000
SLW = 32
NSLICE = D // SLW
NC, NS, L = 2, 16, 16
BL = 512
NB = 62
EPAD = NB * NS * BL
ROWS = B * N
ACC_ROWS = 40960
DUMMY = ROWS


def _sc_body(x2, bi_h, src_h, dst_h, out_h,
             bi_v, src_v, dst_v, gi0, gi1, si0, si1, gb0, gb1, acc):
    c = lax.axis_index("c")
    sub = lax.axis_index("s")
    lanes = lax.iota(jnp.int32, L)
    zero16 = jnp.zeros((L,), jnp.float32)

    for s_local in range(2):
        s = c * 2 + s_local

        @pl.loop(0, BL)
        def _fill(i):
            gb0[i, pl.ds(0, 16)] = zero16
            gb0[i, pl.ds(16, 16)] = zero16

        @pl.loop(0, 5)
        def _zero(k):
            pltpu.sync_copy(gb0, acc.at[pl.ds(sub * 2560 + k * 512, 512)])

        plsc.subcore_barrier()

        @pl.loop(0, NB)
        def _blocks(blk):
            base = (blk * NS + sub) * BL
            pltpu.sync_copy(bi_h.at[pl.ds(base, BL)], bi_v)
            pltpu.sync_copy(src_h.at[pl.ds(base, BL)], src_v)
            pltpu.sync_copy(dst_h.at[pl.ds(base, BL)], dst_v)

            @pl.loop(0, BL // L)
            def _idx(j):
                off = j * L
                b16 = bi_v[pl.ds(off, L)]
                s16 = src_v[pl.ds(off, L)]
                d16 = dst_v[pl.ds(off, L)]
                ls = b16 * N + s16
                ld = b16 * N + d16
                valid = (base + off + lanes) < E
                row = j // 8
                col = (j % 8) * L
                gi0[row, pl.ds(col, L)] = ls * NSLICE + s
                gi1[row, pl.ds(col, L)] = ld * NSLICE + s
                si0[row, pl.ds(col, L)] = jnp.where(valid, ld, DUMMY)
                si1[row, pl.ds(col, L)] = jnp.where(valid, ls, DUMMY)

            @pl.loop(0, BL // 128)
            def _stream(j):
                r = pl.ds(j * 128, 128)
                pltpu.sync_copy(x2.at[gi0.at[j]], gb0.at[r])
                pltpu.sync_copy(x2.at[gi1.at[j]], gb1.at[r])
                pltpu.sync_copy(gb0.at[r], acc.at[si0.at[j]], add=True)
                pltpu.sync_copy(gb1.at[r], acc.at[si1.at[j]], add=True)

        plsc.subcore_barrier()

        @pl.loop(0, 5)
        def _wb(k):
            r0 = sub * 2560 + k * 512
            pltpu.sync_copy(acc.at[pl.ds(r0, 512)], gb0.at[pl.ds(0, 512)])
            pltpu.sync_copy(gb0.at[pl.ds(0, 512)], out_h.at[s, pl.ds(r0, 512)])

        plsc.subcore_barrier()


_sc_call = functools.partial(
    pl.kernel,
    out_type=jax.ShapeDtypeStruct((NSLICE, ACC_ROWS, SLW), jnp.float32),
    mesh=plsc.VectorSubcoreMesh(core_axis_name="c", subcore_axis_name="s"),
    compiler_params=pltpu.CompilerParams(use_tc_tiling_on_sc=False),
    scratch_types=[
        pltpu.VMEM((BL,), jnp.int32),
        pltpu.VMEM((BL,), jnp.int32),
        pltpu.VMEM((BL,), jnp.int32),
        pltpu.VMEM((BL // 128, 128), jnp.int32),
        pltpu.VMEM((BL // 128, 128), jnp.int32),
        pltpu.VMEM((BL // 128, 128), jnp.int32),
        pltpu.VMEM((BL // 128, 128), jnp.int32),
        pltpu.VMEM((BL, SLW), jnp.float32),
        pltpu.VMEM((BL, SLW), jnp.float32),
        pltpu.VMEM_SHARED((ACC_ROWS, SLW), jnp.float32),
    ],
)(_sc_body)


def kernel(x, batch_idx, src, dst):
    x2 = x.reshape(B * N * NSLICE, SLW)
    pad = EPAD - E
    bi_p = jnp.pad(batch_idx.astype(jnp.int32), (0, pad))
    src_p = jnp.pad(src.astype(jnp.int32), (0, pad))
    dst_p = jnp.pad(dst.astype(jnp.int32), (0, pad))
    out_k = _sc_call(x2, bi_p, src_p, dst_p)[:, :ROWS]
    out = out_k.transpose(1, 0, 2).reshape(B, N, D)
    return jnp.concatenate([out, x], axis=2)


# speedup vs baseline: 3.2657x; 1.6324x over previous
import functools

import jax
import jax.numpy as jnp
from jax import lax
from jax.experimental import pallas as pl
from jax.experimental.pallas import tpu as pltpu
from jax.experimental.pallas import tpu_sc as plsc

B, N, D = 4, 10000, 128
E = 500000
SLW = 32
NSLICE = D // SLW
NC, NS, L = 2, 16, 16
BL = 256
NB = 124
NCH = BL // 128
EPAD = NB * NS * BL
ROWS = B * N
ACC_ROWS = 40960
DUMMY = ROWS


def _sc_body(x2, bi_h, src_h, dst_h, out_h, *refs):
    (bi_v0, src_v0, dst_v0, bi_v1, src_v1, dst_v1,
     gi0a, gi1a, si0a, si1a, gi0b, gi1b, si0b, si1b,
     gb0a, gb1a, gb0b, gb1b, acc,
     isem0, isem1, gsem0, gsem1, ssem0, ssem1) = refs
    idx_v = ((bi_v0, src_v0, dst_v0), (bi_v1, src_v1, dst_v1))
    gi = ((gi0a, gi1a), (gi0b, gi1b))
    si = ((si0a, si1a), (si0b, si1b))
    gb = ((gb0a, gb1a), (gb0b, gb1b))
    isem = (isem0, isem1)
    gsem = (gsem0, gsem1)
    ssem = (ssem0, ssem1)
    idx_h = (bi_h, src_h, dst_h)

    c = lax.axis_index("c")
    sub = lax.axis_index("s")
    lanes = lax.iota(jnp.int32, L)
    zero16 = jnp.zeros((L,), jnp.float32)

    def issue_idx(blk, p):
        base = (blk * NS + sub) * BL
        for h, v in zip(idx_h, idx_v[p]):
            pltpu.async_copy(h.at[pl.ds(base, BL)], v, isem[p])

    def wait_idx(p):
        for h, v in zip(idx_h, idx_v[p]):
            pltpu.make_async_copy(h.at[pl.ds(0, BL)], v, isem[p]).wait()

    def compute_idx(blk, p, s):
        base = (blk * NS + sub) * BL
        bi_v, src_v, dst_v = idx_v[p]

        @pl.loop(0, BL // L)
        def _idx(j):
            off = j * L
            b16 = bi_v[pl.ds(off, L)]
            s16 = src_v[pl.ds(off, L)]
            d16 = dst_v[pl.ds(off, L)]
            ls = b16 * N + s16
            ld = b16 * N + d16
            valid = (base + off + lanes) < E
            row = j // 8
            col = (j % 8) * L
            gi[p][0][row, pl.ds(col, L)] = ls * NSLICE + s
            gi[p][1][row, pl.ds(col, L)] = ld * NSLICE + s
            si[p][0][row, pl.ds(col, L)] = jnp.where(valid, ld, DUMMY)
            si[p][1][row, pl.ds(col, L)] = jnp.where(valid, ls, DUMMY)

    def issue_gathers(p):
        descs = []
        for d in range(2):
            for j in range(NCH):
                descs.append(pltpu.async_copy(
                    x2.at[gi[p][d].at[j]],
                    gb[p][d].at[pl.ds(j * 128, 128)], gsem[p]))
        return descs

    def issue_scatters(p):
        for d in range(2):
            for j in range(NCH):
                pltpu.async_copy(gb[p][d].at[pl.ds(j * 128, 128)],
                                 acc.at[si[p][d].at[j]], ssem[p], add=True)

    def drain_scatters(p):
        for d in range(2):
            for j in range(NCH):
                pltpu.make_async_copy(gb[p][d].at[pl.ds(j * 128, 128)],
                                      acc.at[si[p][d].at[j]],
                                      ssem[p]).wait()

    for s_local in range(2):
        s = c * 2 + s_local

        @pl.loop(0, BL)
        def _fill(i):
            gb0a[i, pl.ds(0, 16)] = zero16
            gb0a[i, pl.ds(16, 16)] = zero16
            gb0b[i, pl.ds(0, 16)] = zero16
            gb0b[i, pl.ds(16, 16)] = zero16

        @pl.loop(0, 5)
        def _zero(k):
            pltpu.sync_copy(gb0a, acc.at[pl.ds(sub * 2560 + k * 512, 256)])
            pltpu.sync_copy(gb0b,
                            acc.at[pl.ds(sub * 2560 + k * 512 + 256, 256)])

        plsc.subcore_barrier()

        issue_idx(0, 0)

        @pl.loop(0, NB // 2)
        def _blk2(half):
            for par in range(2):
                blk = half * 2 + par
                p, q = par, 1 - par

                @pl.when(blk + 1 < NB)
                def _prefetch():
                    issue_idx(blk + 1, q)

                @pl.when(blk >= 2)
                def _drain():
                    drain_scatters(p)

                wait_idx(p)
                compute_idx(blk, p, s)
                for desc in issue_gathers(p):
                    desc.wait()
                issue_scatters(p)

        drain_scatters(0)
        drain_scatters(1)
        plsc.subcore_barrier()

        @pl.loop(0, 5)
        def _wb(k):
            r0 = sub * 2560 + k * 512
            pltpu.sync_copy(acc.at[pl.ds(r0, 256)], gb0a)
            pltpu.sync_copy(gb0a, out_h.at[s, pl.ds(r0, 256)])
            pltpu.sync_copy(acc.at[pl.ds(r0 + 256, 256)], gb0b)
            pltpu.sync_copy(gb0b, out_h.at[s, pl.ds(r0 + 256, 256)])

        plsc.subcore_barrier()


_sc_call = functools.partial(
    pl.kernel,
    out_type=jax.ShapeDtypeStruct((NSLICE, ACC_ROWS, SLW), jnp.float32),
    mesh=plsc.VectorSubcoreMesh(core_axis_name="c", subcore_axis_name="s"),
    compiler_params=pltpu.CompilerParams(use_tc_tiling_on_sc=False),
    scratch_types=(
        [pltpu.VMEM((BL,), jnp.int32)] * 6
        + [pltpu.VMEM((NCH, 128), jnp.int32)] * 8
        + [pltpu.VMEM((BL, SLW), jnp.float32)] * 4
        + [pltpu.VMEM_SHARED((ACC_ROWS, SLW), jnp.float32)]
        + [pltpu.SemaphoreType.DMA] * 6
    ),
)(_sc_body)


def kernel(x, batch_idx, src, dst):
    x2 = x.reshape(B * N * NSLICE, SLW)
    pad = EPAD - E
    bi_p = jnp.pad(batch_idx.astype(jnp.int32), (0, pad))
    src_p = jnp.pad(src.astype(jnp.int32), (0, pad))
    dst_p = jnp.pad(dst.astype(jnp.int32), (0, pad))
    out_k = _sc_call(x2, bi_p, src_p, dst_p)[:, :ROWS]
    out = out_k.transpose(1, 0, 2).reshape(B, N, D)
    return jnp.concatenate([out, x], axis=2)


# speedup vs baseline: 3.7571x; 1.1505x over previous
import functools

import jax
import jax.numpy as jnp
from jax import lax
from jax.experimental import pallas as pl
from jax.experimental.pallas import tpu as pltpu
from jax.experimental.pallas import tpu_sc as plsc

B, N, D = 4, 10000, 128
E = 500000
SLW = 32
NSLICE = D // SLW
NC, NS, L = 2, 16, 16
BL = 256
NB = 124
NCH = BL // 128
EPAD = NB * NS * BL
ROWS = B * N
ACC_ROWS = 40960
DUMMY = ROWS


def _sc_body(x2, idx3_h, out_h, *refs):
    (iv0, iv1,
     gi0a, gi1a, si0a, si1a, gi0b, gi1b, si0b, si1b,
     gb0a, gb1a, gb0b, gb1b, xv, acc,
     isem0, isem1, gsem0, gsem1, ssem0, ssem1) = refs
    idx_v = (iv0, iv1)
    gi = ((gi0a, gi1a), (gi0b, gi1b))
    si = ((si0a, si1a), (si0b, si1b))
    gb = ((gb0a, gb1a), (gb0b, gb1b))
    isem = (isem0, isem1)
    gsem = (gsem0, gsem1)
    ssem = (ssem0, ssem1)

    c = lax.axis_index("c")
    sub = lax.axis_index("s")
    lanes = lax.iota(jnp.int32, L)
    zero16 = jnp.zeros((L,), jnp.float32)

    def issue_idx(blk, p):
        base = (blk * NS + sub) * BL
        pltpu.async_copy(idx3_h.at[:, pl.ds(base, BL)], idx_v[p], isem[p])

    def wait_idx(p):
        pltpu.make_async_copy(idx3_h.at[:, pl.ds(0, BL)], idx_v[p],
                              isem[p]).wait()

    def compute_idx(blk, p, s):
        base = (blk * NS + sub) * BL
        v = idx_v[p]

        @pl.loop(0, BL // L)
        def _idx(j):
            off = j * L
            b16 = v[0, pl.ds(off, L)]
            s16 = v[1, pl.ds(off, L)]
            d16 = v[2, pl.ds(off, L)]
            ls = b16 * N + s16
            ld = b16 * N + d16
            valid = (base + off + lanes) < E
            row = j // 8
            col = (j % 8) * L
            gi[p][0][row, pl.ds(col, L)] = ls * NSLICE + s
            gi[p][1][row, pl.ds(col, L)] = ld * NSLICE + s
            si[p][0][row, pl.ds(col, L)] = jnp.where(valid, ld, DUMMY)
            si[p][1][row, pl.ds(col, L)] = jnp.where(valid, ls, DUMMY)

    def issue_gathers(p):
        for d in range(2):
            for j in range(NCH):
                pltpu.async_copy(x2.at[gi[p][d].at[j]],
                                 gb[p][d].at[pl.ds(j * 128, 128)], gsem[p])

    def wait_gathers(p):
        for d in range(2):
            for j in range(NCH):
                pltpu.make_async_copy(x2.at[gi[p][d].at[j]],
                                      gb[p][d].at[pl.ds(j * 128, 128)],
                                      gsem[p]).wait()

    def issue_scatters(p):
        for d in range(2):
            for j in range(NCH):
                pltpu.async_copy(gb[p][d].at[pl.ds(j * 128, 128)],
                                 acc.at[si[p][d].at[j]], ssem[p], add=True)

    def drain_scatters(p):
        for d in range(2):
            for j in range(NCH):
                pltpu.make_async_copy(gb[p][d].at[pl.ds(j * 128, 128)],
                                      acc.at[si[p][d].at[j]],
                                      ssem[p]).wait()

    for s_local in range(2):
        s = c * 2 + s_local

        @pl.loop(0, BL)
        def _fill(i):
            gb0a[i, pl.ds(0, 16)] = zero16
            gb0a[i, pl.ds(16, 16)] = zero16
            gb0b[i, pl.ds(0, 16)] = zero16
            gb0b[i, pl.ds(16, 16)] = zero16

        @pl.loop(0, 5)
        def _zero(k):
            pltpu.sync_copy(gb0a, acc.at[pl.ds(sub * 2560 + k * 512, 256)])
            pltpu.sync_copy(gb0b,
                            acc.at[pl.ds(sub * 2560 + k * 512 + 256, 256)])

        plsc.subcore_barrier()

        issue_idx(0, 0)

        @pl.loop(0, NB // 2)
        def _blk2(half):
            for par in range(2):
                blk = half * 2 + par
                p, q = par, 1 - par

                @pl.when(blk + 1 < NB)
                def _prefetch():
                    issue_idx(blk + 1, q)

                @pl.when(blk >= 2)
                def _drain():
                    drain_scatters(p)

                wait_idx(p)
                compute_idx(blk, p, s)
                issue_gathers(p)

                if par == 1:
                    wait_gathers(q)
                    issue_scatters(q)
                else:
                    @pl.when(half >= 1)
                    def _sc_prev():
                        wait_gathers(q)
                        issue_scatters(q)

        drain_scatters(0)
        wait_gathers(1)
        issue_scatters(1)
        drain_scatters(1)
        plsc.subcore_barrier()

        @pl.loop(0, 20)
        def _wb(k):
            r0 = sub * 2500 + jnp.minimum(k * 128, 2500 - 128)
            rr = pl.ds(r0, 128)
            b128 = pl.ds(0, 128)

            @pl.loop(0, 8)
            def _xidx(j):
                xv[pl.ds(j * 16, 16)] = (r0 + j * 16 + lanes) * NSLICE + s

            pltpu.sync_copy(acc.at[rr], gb0a.at[b128])
            pltpu.sync_copy(gb0a.at[b128], out_h.at[rr, s])
            pltpu.sync_copy(x2.at[xv], gb0b.at[b128])
            pltpu.sync_copy(gb0b.at[b128], out_h.at[rr, NSLICE + s])

        plsc.subcore_barrier()


_sc_call = functools.partial(
    pl.kernel,
    out_type=jax.ShapeDtypeStruct((ROWS, 2 * NSLICE, SLW), jnp.float32),
    mesh=plsc.VectorSubcoreMesh(core_axis_name="c", subcore_axis_name="s"),
    compiler_params=pltpu.CompilerParams(use_tc_tiling_on_sc=False),
    scratch_types=(
        [pltpu.VMEM((3, BL), jnp.int32)] * 2
        + [pltpu.VMEM((NCH, 128), jnp.int32)] * 8
        + [pltpu.VMEM((BL, SLW), jnp.float32)] * 4
        + [pltpu.VMEM((128,), jnp.int32)]
        + [pltpu.VMEM_SHARED((ACC_ROWS, SLW), jnp.float32)]
        + [pltpu.SemaphoreType.DMA] * 6
    ),
)(_sc_body)


def kernel(x, batch_idx, src, dst):
    x2 = x.reshape(B * N * NSLICE, SLW)
    idx3 = jnp.stack([batch_idx.astype(jnp.int32),
                      src.astype(jnp.int32),
                      dst.astype(jnp.int32)])
    idx3 = jnp.pad(idx3, ((0, 0), (0, EPAD - E)))
    out_k = _sc_call(x2, idx3)
    return out_k.reshape(B, N, 2 * D)


# speedup vs baseline: 3.7586x; 1.0004x over previous
import functools

import jax
import jax.numpy as jnp
from jax import lax
from jax.experimental import pallas as pl
from jax.experimental.pallas import tpu as pltpu
from jax.experimental.pallas import tpu_sc as plsc

B, N, D = 4, 10000, 128
E = 500000
SLW = 32
NSLICE = D // SLW
NC, NS, L = 2, 16, 16
BL = 256
NB = 124
NCH = BL // 128
EPAD = NB * NS * BL
ROWS = B * N
ACC_ROWS = 40960
DUMMY = ROWS


def _sc_body(x2, idx3_h, out_h, *refs):
    (iv0, iv1,
     gi0a, gi1a, si0a, si1a, gi0b, gi1b, si0b, si1b,
     gb0a, gb1a, gb0b, gb1b, xv, acc,
     isem0, isem1, gsem0, gsem1, ssem0, ssem1) = refs
    idx_v = (iv0, iv1)
    gi = ((gi0a, gi1a), (gi0b, gi1b))
    si = ((si0a, si1a), (si0b, si1b))
    gb = ((gb0a, gb1a), (gb0b, gb1b))
    isem = (isem0, isem1)
    gsem = (gsem0, gsem1)
    ssem = (ssem0, ssem1)

    c = lax.axis_index("c")
    sub = lax.axis_index("s")
    lanes = lax.iota(jnp.int32, L)
    zero16 = jnp.zeros((L,), jnp.float32)

    def issue_idx(blk, p):
        base = (blk * NS + sub) * BL
        pltpu.async_copy(idx3_h.at[:, pl.ds(base, BL)], idx_v[p], isem[p])

    def wait_idx(p):
        pltpu.make_async_copy(idx3_h.at[:, pl.ds(0, BL)], idx_v[p],
                              isem[p]).wait()

    def compute_idx(blk, p, s):
        base = (blk * NS + sub) * BL
        v = idx_v[p]

        @pl.loop(0, BL // L)
        def _idx(j):
            off = j * L
            b16 = v[0, pl.ds(off, L)]
            s16 = v[1, pl.ds(off, L)]
            d16 = v[2, pl.ds(off, L)]
            ls = b16 * N + s16
            ld = b16 * N + d16
            valid = (base + off + lanes) < E
            row = j // 8
            col = (j % 8) * L
            gi[p][0][pl.ds(off, L)] = ls * NSLICE + s
            gi[p][1][pl.ds(off, L)] = ld * NSLICE + s
            si[p][0][row, pl.ds(col, L)] = jnp.where(valid, ld, DUMMY)
            si[p][1][row, pl.ds(col, L)] = jnp.where(valid, ls, DUMMY)

    def issue_gathers(p):
        for d in range(2):
            pltpu.async_copy(x2.at[gi[p][d]], gb[p][d], gsem[p])

    def wait_gathers(p):
        for d in range(2):
            pltpu.make_async_copy(x2.at[gi[p][d]], gb[p][d], gsem[p]).wait()

    def issue_scatters(p):
        for d in range(2):
            for j in range(NCH):
                pltpu.async_copy(gb[p][d].at[pl.ds(j * 128, 128)],
                                 acc.at[si[p][d].at[j]], ssem[p], add=True)

    def drain_scatters(p):
        for d in range(2):
            for j in range(NCH):
                pltpu.make_async_copy(gb[p][d].at[pl.ds(j * 128, 128)],
                                      acc.at[si[p][d].at[j]],
                                      ssem[p]).wait()

    for s_local in range(2):
        s = c * 2 + s_local

        @pl.loop(0, BL)
        def _fill(i):
            gb0a[i, pl.ds(0, 16)] = zero16
            gb0a[i, pl.ds(16, 16)] = zero16
            gb0b[i, pl.ds(0, 16)] = zero16
            gb0b[i, pl.ds(16, 16)] = zero16

        @pl.loop(0, 5)
        def _zero(k):
            pltpu.sync_copy(gb0a, acc.at[pl.ds(sub * 2560 + k * 512, 256)])
            pltpu.sync_copy(gb0b,
                            acc.at[pl.ds(sub * 2560 + k * 512 + 256, 256)])

        plsc.subcore_barrier()

        issue_idx(0, 0)

        @pl.loop(0, NB // 2)
        def _blk2(half):
            for par in range(2):
                blk = half * 2 + par
                p, q = par, 1 - par

                @pl.when(blk + 1 < NB)
                def _prefetch():
                    issue_idx(blk + 1, q)

                @pl.when(blk >= 2)
                def _drain():
                    drain_scatters(p)

                wait_idx(p)
                compute_idx(blk, p, s)
                issue_gathers(p)

                if par == 1:
                    wait_gathers(q)
                    issue_scatters(q)
                else:
                    @pl.when(half >= 1)
                    def _sc_prev():
                        wait_gathers(q)
                        issue_scatters(q)

        drain_scatters(0)
        wait_gathers(1)
        issue_scatters(1)
        drain_scatters(1)
        plsc.subcore_barrier()

        @pl.loop(0, 20)
        def _wb(k):
            r0 = sub * 2500 + jnp.minimum(k * 128, 2500 - 128)
            rr = pl.ds(r0, 128)
            b128 = pl.ds(0, 128)

            @pl.loop(0, 8)
            def _xidx(j):
                xv[pl.ds(j * 16, 16)] = (r0 + j * 16 + lanes) * NSLICE + s

            pltpu.sync_copy(acc.at[rr], gb0a.at[b128])
            pltpu.sync_copy(gb0a.at[b128], out_h.at[rr, s])
            pltpu.sync_copy(x2.at[xv], gb0b.at[b128])
            pltpu.sync_copy(gb0b.at[b128], out_h.at[rr, NSLICE + s])

        plsc.subcore_barrier()


_sc_call = functools.partial(
    pl.kernel,
    out_type=jax.ShapeDtypeStruct((ROWS, 2 * NSLICE, SLW), jnp.float32),
    mesh=plsc.VectorSubcoreMesh(core_axis_name="c", subcore_axis_name="s"),
    compiler_params=pltpu.CompilerParams(use_tc_tiling_on_sc=False),
    scratch_types=(
        [pltpu.VMEM((3, BL), jnp.int32)] * 2
        + [pltpu.VMEM((BL,), jnp.int32),
           pltpu.VMEM((BL,), jnp.int32),
           pltpu.VMEM((NCH, 128), jnp.int32),
           pltpu.VMEM((NCH, 128), jnp.int32),
           pltpu.VMEM((BL,), jnp.int32),
           pltpu.VMEM((BL,), jnp.int32),
           pltpu.VMEM((NCH, 128), jnp.int32),
           pltpu.VMEM((NCH, 128), jnp.int32)]
        + [pltpu.VMEM((BL, SLW), jnp.float32)] * 4
        + [pltpu.VMEM((128,), jnp.int32)]
        + [pltpu.VMEM_SHARED((ACC_ROWS, SLW), jnp.float32)]
        + [pltpu.SemaphoreType.DMA] * 6
    ),
)(_sc_body)


def kernel(x, batch_idx, src, dst):
    x2 = x.reshape(B * N * NSLICE, SLW)
    idx3 = jnp.stack([batch_idx.astype(jnp.int32),
                      src.astype(jnp.int32),
                      dst.astype(jnp.int32)])
    idx3 = jnp.pad(idx3, ((0, 0), (0, EPAD - E)))
    out_k = _sc_call(x2, idx3)
    return out_k.reshape(B, N, 2 * D)


# speedup vs baseline: 8.8266x; 2.3484x over previous
import functools

import jax
import jax.numpy as jnp
from jax import lax
from jax.experimental import pallas as pl
from jax.experimental.pallas import tpu as pltpu
from jax.experimental.pallas import tpu_sc as plsc

B, N, D = 4, 10000, 128
E = 500000
SLW = 32
NSLICE = D // SLW
NC, NS, L = 2, 16, 16
BL = 256
NB = 124
NCH = BL // 128
EPAD = NB * NS * BL
ROWS = B * N
ACC_ROWS = 40960
DUMMY = ROWS


def _sc_body(x2, bi_h, src_h, dst_h, out_h, *refs):
    (bva0, sva0, dva0, bvb0, svb0, dvb0,
     gi0a, gi1a, si0a, si1a, gi0b, gi1b, si0b, si1b,
     gb0a, gb1a, gb0b, gb1b, xv, acc,
     isem0, isem1, gsem0, gsem1, ssem0, ssem1) = refs
    idx_v = ((bva0, sva0, dva0), (bvb0, svb0, dvb0))
    gi = ((gi0a, gi1a), (gi0b, gi1b))
    si = ((si0a, si1a), (si0b, si1b))
    gb = ((gb0a, gb1a), (gb0b, gb1b))
    isem = (isem0, isem1)
    gsem = (gsem0, gsem1)
    ssem = (ssem0, ssem1)

    c = lax.axis_index("c")
    sub = lax.axis_index("s")
    lanes = lax.iota(jnp.int32, L)
    zero16 = jnp.zeros((L,), jnp.float32)

    def issue_idx(blk, p):
        base = jnp.minimum((blk * NS + sub) * BL, E - BL)
        for h, v in zip((bi_h, src_h, dst_h), idx_v[p]):
            pltpu.async_copy(h.at[pl.ds(base, BL)], v, isem[p])

    def wait_idx(p):
        for h, v in zip((bi_h, src_h, dst_h), idx_v[p]):
            pltpu.make_async_copy(h.at[pl.ds(0, BL)], v, isem[p]).wait()

    def compute_idx(blk, p, s):
        base = (blk * NS + sub) * BL
        base_c = jnp.minimum(base, E - BL)
        bi_v, src_v, dst_v = idx_v[p]

        @pl.loop(0, BL // L)
        def _idx(j):
            off = j * L
            b16 = bi_v[pl.ds(off, L)]
            s16 = src_v[pl.ds(off, L)]
            d16 = dst_v[pl.ds(off, L)]
            ls = b16 * N + s16
            ld = b16 * N + d16
            valid = (base_c + off + lanes) >= base
            row = j // 8
            col = (j % 8) * L
            gi[p][0][pl.ds(off, L)] = ls * NSLICE + s
            gi[p][1][pl.ds(off, L)] = ld * NSLICE + s
            si[p][0][row, pl.ds(col, L)] = jnp.where(valid, ld, DUMMY)
            si[p][1][row, pl.ds(col, L)] = jnp.where(valid, ls, DUMMY)

    def issue_gathers(p):
        for d in range(2):
            pltpu.async_copy(x2.at[gi[p][d]], gb[p][d], gsem[p])

    def wait_gathers(p):
        for d in range(2):
            pltpu.make_async_copy(x2.at[gi[p][d]], gb[p][d], gsem[p]).wait()

    def issue_scatters(p):
        for d in range(2):
            for j in range(NCH):
                pltpu.async_copy(gb[p][d].at[pl.ds(j * 128, 128)],
                                 acc.at[si[p][d].at[j]], ssem[p], add=True)

    def drain_scatters(p):
        for d in range(2):
            for j in range(NCH):
                pltpu.make_async_copy(gb[p][d].at[pl.ds(j * 128, 128)],
                                      acc.at[si[p][d].at[j]],
                                      ssem[p]).wait()

    for s_local in range(2):
        s = c * 2 + s_local

        @pl.loop(0, BL)
        def _fill(i):
            gb0a[i, pl.ds(0, 16)] = zero16
            gb0a[i, pl.ds(16, 16)] = zero16
            gb0b[i, pl.ds(0, 16)] = zero16
            gb0b[i, pl.ds(16, 16)] = zero16

        @pl.loop(0, 5)
        def _zero(k):
            pltpu.sync_copy(gb0a, acc.at[pl.ds(sub * 2560 + k * 512, 256)])
            pltpu.sync_copy(gb0b,
                            acc.at[pl.ds(sub * 2560 + k * 512 + 256, 256)])

        plsc.subcore_barrier()

        issue_idx(0, 0)

        @pl.loop(0, NB // 2)
        def _blk2(half):
            for par in range(2):
                blk = half * 2 + par
                p, q = par, 1 - par

                @pl.when(blk + 1 < NB)
                def _prefetch():
                    issue_idx(blk + 1, q)

                @pl.when(blk >= 2)
                def _drain():
                    drain_scatters(p)

                wait_idx(p)
                compute_idx(blk, p, s)
                issue_gathers(p)

                if par == 1:
                    wait_gathers(q)
                    issue_scatters(q)
                else:
                    @pl.when(half >= 1)
                    def _sc_prev():
                        wait_gathers(q)
                        issue_scatters(q)

        drain_scatters(0)
        wait_gathers(1)
        issue_scatters(1)
        drain_scatters(1)
        plsc.subcore_barrier()

        @pl.loop(0, 20)
        def _wb(k):
            r0 = sub * 2500 + jnp.minimum(k * 128, 2500 - 128)
            rr = pl.ds(r0, 128)
            b128 = pl.ds(0, 128)

            @pl.loop(0, 8)
            def _xidx(j):
                xv[pl.ds(j * 16, 16)] = (r0 + j * 16 + lanes) * NSLICE + s

            pltpu.sync_copy(acc.at[rr], gb0a.at[b128])
            pltpu.sync_copy(gb0a.at[b128], out_h.at[rr, s])
            pltpu.sync_copy(x2.at[xv], gb0b.at[b128])
            pltpu.sync_copy(gb0b.at[b128], out_h.at[rr, NSLICE + s])

        plsc.subcore_barrier()


_sc_call = functools.partial(
    pl.kernel,
    out_type=jax.ShapeDtypeStruct((ROWS, 2 * NSLICE, SLW), jnp.float32),
    mesh=plsc.VectorSubcoreMesh(core_axis_name="c", subcore_axis_name="s"),
    compiler_params=pltpu.CompilerParams(use_tc_tiling_on_sc=False),
    scratch_types=(
        [pltpu.VMEM((BL,), jnp.int32)] * 3
        + [pltpu.VMEM((BL,), jnp.int32)] * 3
        + [pltpu.VMEM((BL,), jnp.int32),
           pltpu.VMEM((BL,), jnp.int32),
           pltpu.VMEM((NCH, 128), jnp.int32),
           pltpu.VMEM((NCH, 128), jnp.int32),
           pltpu.VMEM((BL,), jnp.int32),
           pltpu.VMEM((BL,), jnp.int32),
           pltpu.VMEM((NCH, 128), jnp.int32),
           pltpu.VMEM((NCH, 128), jnp.int32)]
        + [pltpu.VMEM((BL, SLW), jnp.float32)] * 4
        + [pltpu.VMEM((128,), jnp.int32)]
        + [pltpu.VMEM_SHARED((ACC_ROWS, SLW), jnp.float32)]
        + [pltpu.SemaphoreType.DMA] * 6
    ),
)(_sc_body)


def kernel(x, batch_idx, src, dst):
    x2 = x.reshape(B * N * NSLICE, SLW)
    out_k = _sc_call(x2, batch_idx.astype(jnp.int32),
                     src.astype(jnp.int32), dst.astype(jnp.int32))
    return out_k.reshape(B, N, 2 * D)
